# baseline (speedup 1.0000x reference)
import functools
import math

import jax
import jax.numpy as jnp
from jax import lax
from jax.experimental import pallas as pl
from jax.experimental.pallas import tpu as pltpu
from jax.experimental.pallas import tpu_sc as plsc

_N = 10000
_E = 320000
_D = 128
_C = 64
_LAYERS = 4
_ALPHA = 0.5

_NC = 2
_NS = 16
_NW = _NC * _NS
_EPW = _E // _NW
_K = 128
_PAD = 240
_EPWP = _EPW + _PAD
_NCH = _EPWP // _K
_NP = 10240
_RPS = _NP // _NS
_ZR = 64
_TRASH = _NP - 1


def _segsum_body(x_hbm, src_hbm, dst_hbm, out_hbm, acc, sbuf, dbuf, rows, zbuf,
                 gsem, ssem, dsem):
    cid = lax.axis_index("c")
    sid = lax.axis_index("s")
    wid = cid * _NS + sid

    zero = jnp.zeros((16,), jnp.float32)

    def zstore(i, _):
        r = i // (_D // 16)
        c = i % (_D // 16)
        zbuf[r, pl.ds(c * 16, 16)] = zero
        return 0

    lax.fori_loop(0, _ZR * (_D // 16), zstore, 0)

    def zcopy(j, _):
        pltpu.sync_copy(zbuf, acc.at[pl.ds(sid * _RPS + j * _ZR, _ZR)])
        return 0

    lax.fori_loop(0, _RPS // _ZR, zcopy, 0)

    ebase = wid * _EPWP

    def _sload(i, b):
        return pltpu.make_async_copy(src_hbm.at[pl.ds(ebase + i * _K, _K)],
                                     sbuf.at[b], ssem.at[b])

    def _dload(i, b):
        return pltpu.make_async_copy(dst_hbm.at[pl.ds(ebase + i * _K, _K)],
                                     dbuf.at[b], dsem.at[b])

    def _gather(b):
        return pltpu.make_async_copy(x_hbm.at[sbuf.at[b]], rows.at[b],
                                     gsem.at[b])

    _sload(0, 0).start()
    _sload(1, 1).start()
    _dload(0, 0).start()
    _dload(1, 1).start()
    _sload(0, 0).wait()
    _gather(0).start()
    plsc.subcore_barrier()

    def group(g, _):
        for b in range(4):
            i = g * 4 + b
            p = b % 2
            _gather(p).wait()

            @pl.when(i + 2 < _NCH)
            def _():
                _sload(i + 2, p).start()

            @pl.when(i + 1 < _NCH)
            def _():
                _sload(i + 1, 1 - p).wait()
                _gather(1 - p).start()

            _dload(i, p).wait()
            pltpu.sync_copy(rows.at[p], acc.at[dbuf.at[p]], add=True)

            @pl.when(i + 2 < _NCH)
            def _():
                _dload(i + 2, p).start()
        return 0

    lax.fori_loop(0, _NCH // 4, group, 0)
    plsc.subcore_barrier()
    pltpu.sync_copy(acc.at[pl.ds(sid * _RPS, _RPS)],
                    out_hbm.at[cid, pl.ds(sid * _RPS, _RPS)])


def _segsum(xcur, src, dst):
    mesh = plsc.VectorSubcoreMesh(core_axis_name="c", subcore_axis_name="s",
                                  num_cores=_NC, num_subcores=_NS)
    f = pl.kernel(
        _segsum_body,
        out_type=jax.ShapeDtypeStruct((_NC, _NP, _D), jnp.float32),
        mesh=mesh,
        scratch_types=[
            pltpu.VMEM_SHARED((_NP, _D), jnp.float32),
            pltpu.VMEM((2, _K), jnp.int32),
            pltpu.VMEM((2, _K), jnp.int32),
            pltpu.VMEM((2, _K, _D), jnp.float32),
            pltpu.VMEM((_ZR, _D), jnp.float32),
            pltpu.SemaphoreType.DMA((2,)),
            pltpu.SemaphoreType.DMA((2,)),
            pltpu.SemaphoreType.DMA((2,)),
        ],
    )
    return f(xcur, src, dst)


_BR = 1000


def _init_tc(x, w0, b0):
    def body(x_ref, w_ref, b_ref, o_ref):
        h = jnp.dot(x_ref[...], w_ref[...],
                    preferred_element_type=jnp.float32) + b_ref[...]
        o_ref[...] = jnp.maximum(h, 0.0)

    return pl.pallas_call(
        body,
        grid=(_N // _BR,),
        in_specs=[pl.BlockSpec((_BR, _D), lambda i: (i, 0)),
                  pl.BlockSpec((_D, _D), lambda i: (0, 0)),
                  pl.BlockSpec((1, _D), lambda i: (0, 0))],
        out_specs=pl.BlockSpec((_BR, _D), lambda i: (i, 0)),
        out_shape=jax.ShapeDtypeStruct((_N, _D), jnp.float32),
    )(x, w0, b0.reshape(1, _D))


def _layer_tc(parts, x0, xcur, w, beta):
    def body(p_ref, x0_ref, xc_ref, w_ref, o_ref):
        agg = p_ref[0] + p_ref[1]
        t = (1.0 - _ALPHA) * agg + _ALPHA * x0_ref[...]
        out = (1.0 - beta) * t + beta * jnp.dot(
            t, w_ref[...], preferred_element_type=jnp.float32)
        o_ref[...] = jnp.maximum(out + xc_ref[...], 0.0)

    return pl.pallas_call(
        body,
        grid=(_N // _BR,),
        in_specs=[pl.BlockSpec((_NC, _BR, _D), lambda i: (0, i, 0)),
                  pl.BlockSpec((_BR, _D), lambda i: (i, 0)),
                  pl.BlockSpec((_BR, _D), lambda i: (i, 0)),
                  pl.BlockSpec((_D, _D), lambda i: (0, 0))],
        out_specs=pl.BlockSpec((_BR, _D), lambda i: (i, 0)),
        out_shape=jax.ShapeDtypeStruct((_N, _D), jnp.float32),
    )(parts, x0, xcur, w)


def _final_tc(xcur, w1, b1):
    def body(x_ref, w_ref, b_ref, o_ref):
        logits = jnp.dot(x_ref[...], w_ref[...],
                         preferred_element_type=jnp.float32) + b_ref[...]
        m = jnp.max(logits, axis=1, keepdims=True)
        z = logits - m
        lse = jnp.log(jnp.sum(jnp.exp(z), axis=1, keepdims=True))
        o_ref[...] = z - lse

    return pl.pallas_call(
        body,
        grid=(_N // _BR,),
        in_specs=[pl.BlockSpec((_BR, _D), lambda i: (i, 0)),
                  pl.BlockSpec((_D, _C), lambda i: (0, 0)),
                  pl.BlockSpec((1, _C), lambda i: (0, 0))],
        out_specs=pl.BlockSpec((_BR, _C), lambda i: (i, 0)),
        out_shape=jax.ShapeDtypeStruct((_N, _C), jnp.float32),
    )(xcur, w1, b1.reshape(1, _C))


def kernel(x, edge_index, w0, b0, conv_w, w1, b1):
    e = edge_index.reshape(2, _NW, _EPW)
    src = jnp.concatenate(
        [e[0], jnp.zeros((_NW, _PAD), jnp.int32)], axis=1).reshape(-1)
    dst = jnp.concatenate(
        [e[1], jnp.full((_NW, _PAD), _TRASH, jnp.int32)], axis=1).reshape(-1)
    h = _init_tc(x, w0, b0)
    x0 = h
    xcur = h
    for layer in range(_LAYERS):
        beta = math.log(1.0 / (layer + 1) + 1.0)
        parts = _segsum(xcur, src, dst)
        xcur = _layer_tc(parts, x0, xcur, conv_w[layer], beta)
    return _final_tc(xcur, w1, b1)


# speedup vs baseline: 1.0447x; 1.0447x over previous
import functools
import math

import jax
import jax.numpy as jnp
from jax import lax
from jax.experimental import pallas as pl
from jax.experimental.pallas import tpu as pltpu
from jax.experimental.pallas import tpu_sc as plsc

_N = 10000
_E = 320000
_D = 128
_C = 64
_LAYERS = 4
_ALPHA = 0.5

_NC = 2
_NS = 16
_NW = _NC * _NS
_EPW = _E // _NW
_K = 32
_NB = 8
_LA = _NB // 2
_PAD = 240
_EPWP = _EPW + _PAD
_NCH = _EPWP // _K
_NP = 10240
_RPS = _NP // _NS
_ZR = 64
_TRASH = _NP - 1


def _segsum_body(x_hbm, src_hbm, dst_hbm, out_hbm, acc, sbuf, dbuf, rows, zbuf,
                 gsem, ssem, dsem):
    cid = lax.axis_index("c")
    sid = lax.axis_index("s")
    wid = cid * _NS + sid

    zero = jnp.zeros((16,), jnp.float32)

    def zstore(i, _):
        r = i // (_D // 16)
        c = i % (_D // 16)
        zbuf[r, pl.ds(c * 16, 16)] = zero
        return 0

    lax.fori_loop(0, _ZR * (_D // 16), zstore, 0)

    def zcopy(j, _):
        pltpu.sync_copy(zbuf, acc.at[pl.ds(sid * _RPS + j * _ZR, _ZR)])
        return 0

    lax.fori_loop(0, _RPS // _ZR, zcopy, 0)

    ebase = wid * _EPWP

    def _sload(i, b):
        return pltpu.make_async_copy(src_hbm.at[pl.ds(ebase + i * _K, _K)],
                                     sbuf.at[b], ssem.at[b])

    def _dload(i, b):
        return pltpu.make_async_copy(dst_hbm.at[pl.ds(ebase + i * _K, _K)],
                                     dbuf.at[b], dsem.at[b])

    def _gather(b):
        return pltpu.make_async_copy(x_hbm.at[sbuf.at[b]], rows.at[b],
                                     gsem.at[b])

    for b in range(_NB):
        _sload(b, b).start()
        _dload(b, b).start()
    for b in range(_LA):
        _sload(b, b).wait()
        _gather(b).start()
    plsc.subcore_barrier()

    def group(g, _):
        for b in range(_NB):
            i = g * _NB + b
            b2 = (b + _LA) % _NB
            _gather(b).wait()

            @pl.when(i + _NB < _NCH)
            def _():
                _sload(i + _NB, b).start()

            _dload(i, b).wait()
            pltpu.sync_copy(rows.at[b], acc.at[dbuf.at[b]], add=True)

            @pl.when(i + _NB < _NCH)
            def _():
                _dload(i + _NB, b).start()

            @pl.when(i + _LA < _NCH)
            def _():
                _sload(i + _LA, b2).wait()
                _gather(b2).start()
        return 0

    lax.fori_loop(0, _NCH // _NB, group, 0)
    plsc.subcore_barrier()
    pltpu.sync_copy(acc.at[pl.ds(sid * _RPS, _RPS)],
                    out_hbm.at[cid, pl.ds(sid * _RPS, _RPS)])


def _segsum(xcur, src, dst):
    mesh = plsc.VectorSubcoreMesh(core_axis_name="c", subcore_axis_name="s",
                                  num_cores=_NC, num_subcores=_NS)
    f = pl.kernel(
        _segsum_body,
        out_type=jax.ShapeDtypeStruct((_NC, _NP, _D), jnp.float32),
        mesh=mesh,
        scratch_types=[
            pltpu.VMEM_SHARED((_NP, _D), jnp.float32),
            pltpu.VMEM((_NB, _K), jnp.int32),
            pltpu.VMEM((_NB, _K), jnp.int32),
            pltpu.VMEM((_NB, _K, _D), jnp.float32),
            pltpu.VMEM((_ZR, _D), jnp.float32),
            pltpu.SemaphoreType.DMA((_NB,)),
            pltpu.SemaphoreType.DMA((_NB,)),
            pltpu.SemaphoreType.DMA((_NB,)),
        ],
    )
    return f(xcur, src, dst)


_BR = 1000


def _init_tc(x, w0, b0):
    def body(x_ref, w_ref, b_ref, o_ref):
        h = jnp.dot(x_ref[...], w_ref[...],
                    preferred_element_type=jnp.float32) + b_ref[...]
        o_ref[...] = jnp.maximum(h, 0.0)

    return pl.pallas_call(
        body,
        grid=(_N // _BR,),
        in_specs=[pl.BlockSpec((_BR, _D), lambda i: (i, 0)),
                  pl.BlockSpec((_D, _D), lambda i: (0, 0)),
                  pl.BlockSpec((1, _D), lambda i: (0, 0))],
        out_specs=pl.BlockSpec((_BR, _D), lambda i: (i, 0)),
        out_shape=jax.ShapeDtypeStruct((_N, _D), jnp.float32),
    )(x, w0, b0.reshape(1, _D))


def _layer_tc(parts, x0, xcur, w, beta):
    def body(p_ref, x0_ref, xc_ref, w_ref, o_ref):
        agg = p_ref[0] + p_ref[1]
        t = (1.0 - _ALPHA) * agg + _ALPHA * x0_ref[...]
        out = (1.0 - beta) * t + beta * jnp.dot(
            t, w_ref[...], preferred_element_type=jnp.float32)
        o_ref[...] = jnp.maximum(out + xc_ref[...], 0.0)

    return pl.pallas_call(
        body,
        grid=(_N // _BR,),
        in_specs=[pl.BlockSpec((_NC, _BR, _D), lambda i: (0, i, 0)),
                  pl.BlockSpec((_BR, _D), lambda i: (i, 0)),
                  pl.BlockSpec((_BR, _D), lambda i: (i, 0)),
                  pl.BlockSpec((_D, _D), lambda i: (0, 0))],
        out_specs=pl.BlockSpec((_BR, _D), lambda i: (i, 0)),
        out_shape=jax.ShapeDtypeStruct((_N, _D), jnp.float32),
    )(parts, x0, xcur, w)


def _final_tc(xcur, w1, b1):
    def body(x_ref, w_ref, b_ref, o_ref):
        logits = jnp.dot(x_ref[...], w_ref[...],
                         preferred_element_type=jnp.float32) + b_ref[...]
        m = jnp.max(logits, axis=1, keepdims=True)
        z = logits - m
        lse = jnp.log(jnp.sum(jnp.exp(z), axis=1, keepdims=True))
        o_ref[...] = z - lse

    return pl.pallas_call(
        body,
        grid=(_N // _BR,),
        in_specs=[pl.BlockSpec((_BR, _D), lambda i: (i, 0)),
                  pl.BlockSpec((_D, _C), lambda i: (0, 0)),
                  pl.BlockSpec((1, _C), lambda i: (0, 0))],
        out_specs=pl.BlockSpec((_BR, _C), lambda i: (i, 0)),
        out_shape=jax.ShapeDtypeStruct((_N, _C), jnp.float32),
    )(xcur, w1, b1.reshape(1, _C))


def kernel(x, edge_index, w0, b0, conv_w, w1, b1):
    e = edge_index.reshape(2, _NW, _EPW)
    src = jnp.concatenate(
        [e[0], jnp.zeros((_NW, _PAD), jnp.int32)], axis=1).reshape(-1)
    dst = jnp.concatenate(
        [e[1], jnp.full((_NW, _PAD), _TRASH, jnp.int32)], axis=1).reshape(-1)
    h = _init_tc(x, w0, b0)
    x0 = h
    xcur = h
    for layer in range(_LAYERS):
        beta = math.log(1.0 / (layer + 1) + 1.0)
        parts = _segsum(xcur, src, dst)
        xcur = _layer_tc(parts, x0, xcur, conv_w[layer], beta)
    return _final_tc(xcur, w1, b1)


# speedup vs baseline: 1.4781x; 1.4149x over previous
import functools
import math

import jax
import jax.numpy as jnp
from jax import lax
from jax.experimental import pallas as pl
from jax.experimental.pallas import tpu as pltpu
from jax.experimental.pallas import tpu_sc as plsc

_N = 10000
_E = 320000
_D = 128
_C = 64
_LAYERS = 4
_ALPHA = 0.5

_NC = 2
_NS = 16
_NW = _NC * _NS
_EPW = _E // _NW
_K = 112
_PAD = 80
_EPWP = _EPW + _PAD
_NCH = _EPWP // _K
_NP = 10240
_RPS = _NP // _NS
_ZR = 64
_TRASH = _NP - 1


def _segsum_body(x_hbm, src_hbm, dst_hbm, out_hbm, acc, sbufa, sbufb, dbufa,
                 dbufb, rows, zbuf, gsem, ssem, dsem):
    cid = lax.axis_index("c")
    sid = lax.axis_index("s")
    wid = cid * _NS + sid

    zero = jnp.zeros((16,), jnp.float32)

    def zstore(i, _):
        r = i // (_D // 16)
        c = i % (_D // 16)
        zbuf[r, pl.ds(c * 16, 16)] = zero
        return 0

    lax.fori_loop(0, _ZR * (_D // 16), zstore, 0)

    def zcopy(j, _):
        pltpu.sync_copy(zbuf, acc.at[pl.ds(sid * _RPS + j * _ZR, _ZR)])
        return 0

    lax.fori_loop(0, _RPS // _ZR, zcopy, 0)

    ebase = wid * _EPWP

    def _sload(i, buf, p):
        return pltpu.make_async_copy(src_hbm.at[pl.ds(ebase + i * _K, _K)],
                                     buf, ssem.at[p])

    def _dload(i, buf, p):
        return pltpu.make_async_copy(dst_hbm.at[pl.ds(ebase + i * _K, _K)],
                                     buf, dsem.at[p])

    _sload(0, sbufa, 0).start()
    _dload(0, dbufa, 0).start()
    _sload(1, sbufb, 1).start()
    _dload(1, dbufb, 1).start()
    plsc.subcore_barrier()

    def group(g, _):
        for p, (sb, db) in enumerate(((sbufa, dbufa), (sbufb, dbufb))):
            i = g * 2 + p
            _sload(i, sb, p).wait()
            pltpu.async_copy(x_hbm.at[sb], rows, gsem).wait()
            _dload(i, db, p).wait()
            pltpu.sync_copy(rows, acc.at[db], add=True)

            @pl.when(i + 2 < _NCH)
            def _():
                _sload(i + 2, sb, p).start()
                _dload(i + 2, db, p).start()
        return 0

    lax.fori_loop(0, _NCH // 2, group, 0)
    plsc.subcore_barrier()
    pltpu.sync_copy(acc.at[pl.ds(sid * _RPS, _RPS)],
                    out_hbm.at[cid, pl.ds(sid * _RPS, _RPS)])


def _segsum(xcur, src, dst):
    mesh = plsc.VectorSubcoreMesh(core_axis_name="c", subcore_axis_name="s",
                                  num_cores=_NC, num_subcores=_NS)
    f = pl.kernel(
        _segsum_body,
        out_type=jax.ShapeDtypeStruct((_NC, _NP, _D), jnp.float32),
        mesh=mesh,
        scratch_types=[
            pltpu.VMEM_SHARED((_NP, _D), jnp.float32),
            pltpu.VMEM((_K,), jnp.int32),
            pltpu.VMEM((_K,), jnp.int32),
            pltpu.VMEM((_K,), jnp.int32),
            pltpu.VMEM((_K,), jnp.int32),
            pltpu.VMEM((_K, _D), jnp.float32),
            pltpu.VMEM((_ZR, _D), jnp.float32),
            pltpu.SemaphoreType.DMA,
            pltpu.SemaphoreType.DMA((2,)),
            pltpu.SemaphoreType.DMA((2,)),
        ],
    )
    return f(xcur, src, dst)


_BR = 1000


def _init_tc(x, w0, b0):
    def body(x_ref, w_ref, b_ref, o_ref):
        h = jnp.dot(x_ref[...], w_ref[...],
                    preferred_element_type=jnp.float32) + b_ref[...]
        o_ref[...] = jnp.maximum(h, 0.0)

    return pl.pallas_call(
        body,
        grid=(_N // _BR,),
        in_specs=[pl.BlockSpec((_BR, _D), lambda i: (i, 0)),
                  pl.BlockSpec((_D, _D), lambda i: (0, 0)),
                  pl.BlockSpec((1, _D), lambda i: (0, 0))],
        out_specs=pl.BlockSpec((_BR, _D), lambda i: (i, 0)),
        out_shape=jax.ShapeDtypeStruct((_N, _D), jnp.float32),
    )(x, w0, b0.reshape(1, _D))


def _layer_tc(parts, x0, xcur, w, beta):
    def body(p_ref, x0_ref, xc_ref, w_ref, o_ref):
        agg = p_ref[0] + p_ref[1]
        t = (1.0 - _ALPHA) * agg + _ALPHA * x0_ref[...]
        out = (1.0 - beta) * t + beta * jnp.dot(
            t, w_ref[...], preferred_element_type=jnp.float32)
        o_ref[...] = jnp.maximum(out + xc_ref[...], 0.0)

    return pl.pallas_call(
        body,
        grid=(_N // _BR,),
        in_specs=[pl.BlockSpec((_NC, _BR, _D), lambda i: (0, i, 0)),
                  pl.BlockSpec((_BR, _D), lambda i: (i, 0)),
                  pl.BlockSpec((_BR, _D), lambda i: (i, 0)),
                  pl.BlockSpec((_D, _D), lambda i: (0, 0))],
        out_specs=pl.BlockSpec((_BR, _D), lambda i: (i, 0)),
        out_shape=jax.ShapeDtypeStruct((_N, _D), jnp.float32),
    )(parts, x0, xcur, w)


def _final_tc(xcur, w1, b1):
    def body(x_ref, w_ref, b_ref, o_ref):
        logits = jnp.dot(x_ref[...], w_ref[...],
                         preferred_element_type=jnp.float32) + b_ref[...]
        m = jnp.max(logits, axis=1, keepdims=True)
        z = logits - m
        lse = jnp.log(jnp.sum(jnp.exp(z), axis=1, keepdims=True))
        o_ref[...] = z - lse

    return pl.pallas_call(
        body,
        grid=(_N // _BR,),
        in_specs=[pl.BlockSpec((_BR, _D), lambda i: (i, 0)),
                  pl.BlockSpec((_D, _C), lambda i: (0, 0)),
                  pl.BlockSpec((1, _C), lambda i: (0, 0))],
        out_specs=pl.BlockSpec((_BR, _C), lambda i: (i, 0)),
        out_shape=jax.ShapeDtypeStruct((_N, _C), jnp.float32),
    )(xcur, w1, b1.reshape(1, _C))


def kernel(x, edge_index, w0, b0, conv_w, w1, b1):
    e = edge_index.reshape(2, _NW, _EPW)
    src = jnp.concatenate(
        [e[0], jnp.zeros((_NW, _PAD), jnp.int32)], axis=1).reshape(-1)
    dst = jnp.concatenate(
        [e[1], jnp.full((_NW, _PAD), _TRASH, jnp.int32)], axis=1).reshape(-1)
    h = _init_tc(x, w0, b0)
    x0 = h
    xcur = h
    for layer in range(_LAYERS):
        beta = math.log(1.0 / (layer + 1) + 1.0)
        parts = _segsum(xcur, src, dst)
        xcur = _layer_tc(parts, x0, xcur, conv_w[layer], beta)
    return _final_tc(xcur, w1, b1)


# speedup vs baseline: 1.7484x; 1.1829x over previous
import functools
import math

import jax
import jax.numpy as jnp
from jax import lax
from jax.experimental import pallas as pl
from jax.experimental.pallas import tpu as pltpu
from jax.experimental.pallas import tpu_sc as plsc

_N = 10000
_E = 320000
_D = 128
_C = 64
_LAYERS = 4
_ALPHA = 0.5

_NC = 2
_NS = 16
_NW = _NC * _NS
_EPW = _E // _NW
_K = 112
_PAD = 80
_EPWP = _EPW + _PAD
_NCH = _EPWP // _K
_NP = 10240
_RPS = _NP // _NS
_ZR = 64
_TRASH = _NP - 1


def _segsum_body(x_hbm, src_hbm, dst_hbm, out_hbm, acc, sbufa, sbufb, dbufa,
                 dbufb, rowsa, rowsb, zbuf, gsem, ssem, dsem):
    cid = lax.axis_index("c")
    sid = lax.axis_index("s")
    wid = cid * _NS + sid

    zero = jnp.zeros((16,), jnp.float32)

    def zstore(i, _):
        r = i // (_D // 16)
        c = i % (_D // 16)
        zbuf[r, pl.ds(c * 16, 16)] = zero
        return 0

    lax.fori_loop(0, _ZR * (_D // 16), zstore, 0)

    def zcopy(j, _):
        pltpu.sync_copy(zbuf, acc.at[pl.ds(sid * _RPS + j * _ZR, _ZR)])
        return 0

    lax.fori_loop(0, _RPS // _ZR, zcopy, 0)

    ebase = wid * _EPWP

    def _sload(i, buf, p):
        return pltpu.make_async_copy(src_hbm.at[pl.ds(ebase + i * _K, _K)],
                                     buf, ssem.at[p])

    def _dload(i, buf, p):
        return pltpu.make_async_copy(dst_hbm.at[pl.ds(ebase + i * _K, _K)],
                                     buf, dsem.at[p])

    def _gather(sb, rw, p):
        return pltpu.make_async_copy(x_hbm.at[sb], rw, gsem.at[p])

    _sload(0, sbufa, 0).start()
    _dload(0, dbufa, 0).start()
    _sload(1, sbufb, 1).start()
    _dload(1, dbufb, 1).start()
    _sload(0, sbufa, 0).wait()
    _gather(sbufa, rowsa, 0).start()
    plsc.subcore_barrier()

    def group(g, _):
        ab = ((sbufa, dbufa, rowsa), (sbufb, dbufb, rowsb))
        for p, (sb, db, rw) in enumerate(ab):
            i = g * 2 + p
            sb2, db2, rw2 = ab[1 - p]
            _gather(sb, rw, p).wait()

            @pl.when(i + 1 < _NCH)
            def _():
                _sload(i + 1, sb2, 1 - p).wait()
                _gather(sb2, rw2, 1 - p).start()

            _dload(i, db, p).wait()
            pltpu.sync_copy(rw, acc.at[db], add=True)

            @pl.when(i + 2 < _NCH)
            def _():
                _sload(i + 2, sb, p).start()
                _dload(i + 2, db, p).start()
        return 0

    lax.fori_loop(0, _NCH // 2, group, 0)
    plsc.subcore_barrier()
    pltpu.sync_copy(acc.at[pl.ds(sid * _RPS, _RPS)],
                    out_hbm.at[cid, pl.ds(sid * _RPS, _RPS)])


def _segsum(xcur, src, dst):
    mesh = plsc.VectorSubcoreMesh(core_axis_name="c", subcore_axis_name="s",
                                  num_cores=_NC, num_subcores=_NS)
    f = pl.kernel(
        _segsum_body,
        out_type=jax.ShapeDtypeStruct((_NC, _NP, _D), jnp.float32),
        mesh=mesh,
        scratch_types=[
            pltpu.VMEM_SHARED((_NP, _D), jnp.float32),
            pltpu.VMEM((_K,), jnp.int32),
            pltpu.VMEM((_K,), jnp.int32),
            pltpu.VMEM((_K,), jnp.int32),
            pltpu.VMEM((_K,), jnp.int32),
            pltpu.VMEM((_K, _D), jnp.float32),
            pltpu.VMEM((_K, _D), jnp.float32),
            pltpu.VMEM((_ZR, _D), jnp.float32),
            pltpu.SemaphoreType.DMA((2,)),
            pltpu.SemaphoreType.DMA((2,)),
            pltpu.SemaphoreType.DMA((2,)),
        ],
    )
    return f(xcur, src, dst)


_BR = 1000


def _init_tc(x, w0, b0):
    def body(x_ref, w_ref, b_ref, o_ref):
        h = jnp.dot(x_ref[...], w_ref[...],
                    preferred_element_type=jnp.float32) + b_ref[...]
        o_ref[...] = jnp.maximum(h, 0.0)

    return pl.pallas_call(
        body,
        grid=(_N // _BR,),
        in_specs=[pl.BlockSpec((_BR, _D), lambda i: (i, 0)),
                  pl.BlockSpec((_D, _D), lambda i: (0, 0)),
                  pl.BlockSpec((1, _D), lambda i: (0, 0))],
        out_specs=pl.BlockSpec((_BR, _D), lambda i: (i, 0)),
        out_shape=jax.ShapeDtypeStruct((_N, _D), jnp.float32),
    )(x, w0, b0.reshape(1, _D))


def _layer_tc(parts, x0, xcur, w, beta):
    def body(p_ref, x0_ref, xc_ref, w_ref, o_ref):
        agg = p_ref[0] + p_ref[1]
        t = (1.0 - _ALPHA) * agg + _ALPHA * x0_ref[...]
        out = (1.0 - beta) * t + beta * jnp.dot(
            t, w_ref[...], preferred_element_type=jnp.float32)
        o_ref[...] = jnp.maximum(out + xc_ref[...], 0.0)

    return pl.pallas_call(
        body,
        grid=(_N // _BR,),
        in_specs=[pl.BlockSpec((_NC, _BR, _D), lambda i: (0, i, 0)),
                  pl.BlockSpec((_BR, _D), lambda i: (i, 0)),
                  pl.BlockSpec((_BR, _D), lambda i: (i, 0)),
                  pl.BlockSpec((_D, _D), lambda i: (0, 0))],
        out_specs=pl.BlockSpec((_BR, _D), lambda i: (i, 0)),
        out_shape=jax.ShapeDtypeStruct((_N, _D), jnp.float32),
    )(parts, x0, xcur, w)


def _final_tc(xcur, w1, b1):
    def body(x_ref, w_ref, b_ref, o_ref):
        logits = jnp.dot(x_ref[...], w_ref[...],
                         preferred_element_type=jnp.float32) + b_ref[...]
        m = jnp.max(logits, axis=1, keepdims=True)
        z = logits - m
        lse = jnp.log(jnp.sum(jnp.exp(z), axis=1, keepdims=True))
        o_ref[...] = z - lse

    return pl.pallas_call(
        body,
        grid=(_N // _BR,),
        in_specs=[pl.BlockSpec((_BR, _D), lambda i: (i, 0)),
                  pl.BlockSpec((_D, _C), lambda i: (0, 0)),
                  pl.BlockSpec((1, _C), lambda i: (0, 0))],
        out_specs=pl.BlockSpec((_BR, _C), lambda i: (i, 0)),
        out_shape=jax.ShapeDtypeStruct((_N, _C), jnp.float32),
    )(xcur, w1, b1.reshape(1, _C))


def kernel(x, edge_index, w0, b0, conv_w, w1, b1):
    e = edge_index.reshape(2, _NW, _EPW)
    src = jnp.concatenate(
        [e[0], jnp.zeros((_NW, _PAD), jnp.int32)], axis=1).reshape(-1)
    dst = jnp.concatenate(
        [e[1], jnp.full((_NW, _PAD), _TRASH, jnp.int32)], axis=1).reshape(-1)
    h = _init_tc(x, w0, b0)
    x0 = h
    xcur = h
    for layer in range(_LAYERS):
        beta = math.log(1.0 / (layer + 1) + 1.0)
        parts = _segsum(xcur, src, dst)
        xcur = _layer_tc(parts, x0, xcur, conv_w[layer], beta)
    return _final_tc(xcur, w1, b1)


# speedup vs baseline: 1.8065x; 1.0332x over previous
import functools
import math

import jax
import jax.numpy as jnp
from jax import lax
from jax.experimental import pallas as pl
from jax.experimental.pallas import tpu as pltpu
from jax.experimental.pallas import tpu_sc as plsc

_N = 10000
_E = 320000
_D = 128
_C = 64
_LAYERS = 4
_ALPHA = 0.5

_NC = 2
_NS = 16
_NW = _NC * _NS
_EPW = _E // _NW
_K = 96
_PAD = 80
_EPWP = _EPW + _PAD
_NCH = _EPWP // _K
_NP = 10240
_RPS = _NP // _NS
_ZR = 64
_TRASH = _NP - 1


def _segsum_body(x_hbm, src_hbm, dst_hbm, out_hbm, acc, sbufs, dbufs, rows,
                 gsem, ssem, dsem):
    cid = lax.axis_index("c")
    sid = lax.axis_index("s")
    wid = cid * _NS + sid

    zero = jnp.zeros((16,), jnp.float32)
    z0 = rows[0]

    def zstore(i, _):
        r = i // (_D // 16)
        c = i % (_D // 16)
        z0[r, pl.ds(c * 16, 16)] = zero
        return 0

    lax.fori_loop(0, _K * (_D // 16), zstore, 0)

    def zcopy(j, _):
        pltpu.sync_copy(z0, acc.at[pl.ds(sid * _RPS + j * _K, _K)])
        return 0

    lax.fori_loop(0, _RPS // _K, zcopy, 0)
    pltpu.sync_copy(z0.at[pl.ds(0, _RPS % _K)],
                    acc.at[pl.ds(sid * _RPS + (_RPS // _K) * _K, _RPS % _K)])

    ebase = wid * _EPWP

    def _sload(i, buf, p):
        return pltpu.make_async_copy(src_hbm.at[pl.ds(ebase + i * _K, _K)],
                                     buf, ssem.at[p])

    def _dload(i, buf, p):
        return pltpu.make_async_copy(dst_hbm.at[pl.ds(ebase + i * _K, _K)],
                                     buf, dsem.at[p])

    def _gather(i, p):
        return pltpu.make_async_copy(x_hbm.at[sbufs[p]], rows[p], gsem.at[p])

    for p in range(3):
        _sload(p, sbufs[p], p).start()
        _dload(p, dbufs[p], p).start()
    for p in range(2):
        _sload(p, sbufs[p], p).wait()
        _gather(p, p).start()
    plsc.subcore_barrier()

    def group(g, _):
        for p in range(3):
            i = g * 3 + p
            p2 = (p + 2) % 3
            _gather(i, p).wait()

            @pl.when(i + 2 < _NCH)
            def _():
                _sload(i + 2, sbufs[p2], p2).wait()
                _gather(i + 2, p2).start()

            _dload(i, dbufs[p], p).wait()
            pltpu.sync_copy(rows[p], acc.at[dbufs[p]], add=True)

            @pl.when(i + 3 < _NCH)
            def _():
                _sload(i + 3, sbufs[p], p).start()
                _dload(i + 3, dbufs[p], p).start()
        return 0

    lax.fori_loop(0, _NCH // 3, group, 0)
    plsc.subcore_barrier()
    pltpu.sync_copy(acc.at[pl.ds(sid * _RPS, _RPS)],
                    out_hbm.at[cid, pl.ds(sid * _RPS, _RPS)])


def _segsum(xcur, src, dst):
    mesh = plsc.VectorSubcoreMesh(core_axis_name="c", subcore_axis_name="s",
                                  num_cores=_NC, num_subcores=_NS)
    f = pl.kernel(
        _segsum_body,
        out_type=jax.ShapeDtypeStruct((_NC, _NP, _D), jnp.float32),
        mesh=mesh,
        scratch_types=[
            pltpu.VMEM_SHARED((_NP, _D), jnp.float32),
            [pltpu.VMEM((_K,), jnp.int32)] * 3,
            [pltpu.VMEM((_K,), jnp.int32)] * 3,
            [pltpu.VMEM((_K, _D), jnp.float32)] * 3,
            pltpu.SemaphoreType.DMA((3,)),
            pltpu.SemaphoreType.DMA((3,)),
            pltpu.SemaphoreType.DMA((3,)),
        ],
    )
    return f(xcur, src, dst)


_BR = 1000


def _init_tc(x, w0, b0):
    def body(x_ref, w_ref, b_ref, o_ref):
        h = jnp.dot(x_ref[...], w_ref[...],
                    preferred_element_type=jnp.float32) + b_ref[...]
        o_ref[...] = jnp.maximum(h, 0.0)

    return pl.pallas_call(
        body,
        grid=(_N // _BR,),
        in_specs=[pl.BlockSpec((_BR, _D), lambda i: (i, 0)),
                  pl.BlockSpec((_D, _D), lambda i: (0, 0)),
                  pl.BlockSpec((1, _D), lambda i: (0, 0))],
        out_specs=pl.BlockSpec((_BR, _D), lambda i: (i, 0)),
        out_shape=jax.ShapeDtypeStruct((_N, _D), jnp.float32),
    )(x, w0, b0.reshape(1, _D))


def _layer_tc(parts, x0, xcur, w, beta):
    def body(p_ref, x0_ref, xc_ref, w_ref, o_ref):
        agg = p_ref[0] + p_ref[1]
        t = (1.0 - _ALPHA) * agg + _ALPHA * x0_ref[...]
        out = (1.0 - beta) * t + beta * jnp.dot(
            t, w_ref[...], preferred_element_type=jnp.float32)
        o_ref[...] = jnp.maximum(out + xc_ref[...], 0.0)

    return pl.pallas_call(
        body,
        grid=(_N // _BR,),
        in_specs=[pl.BlockSpec((_NC, _BR, _D), lambda i: (0, i, 0)),
                  pl.BlockSpec((_BR, _D), lambda i: (i, 0)),
                  pl.BlockSpec((_BR, _D), lambda i: (i, 0)),
                  pl.BlockSpec((_D, _D), lambda i: (0, 0))],
        out_specs=pl.BlockSpec((_BR, _D), lambda i: (i, 0)),
        out_shape=jax.ShapeDtypeStruct((_N, _D), jnp.float32),
    )(parts, x0, xcur, w)


def _final_tc(xcur, w1, b1):
    def body(x_ref, w_ref, b_ref, o_ref):
        logits = jnp.dot(x_ref[...], w_ref[...],
                         preferred_element_type=jnp.float32) + b_ref[...]
        m = jnp.max(logits, axis=1, keepdims=True)
        z = logits - m
        lse = jnp.log(jnp.sum(jnp.exp(z), axis=1, keepdims=True))
        o_ref[...] = z - lse

    return pl.pallas_call(
        body,
        grid=(_N // _BR,),
        in_specs=[pl.BlockSpec((_BR, _D), lambda i: (i, 0)),
                  pl.BlockSpec((_D, _C), lambda i: (0, 0)),
                  pl.BlockSpec((1, _C), lambda i: (0, 0))],
        out_specs=pl.BlockSpec((_BR, _C), lambda i: (i, 0)),
        out_shape=jax.ShapeDtypeStruct((_N, _C), jnp.float32),
    )(xcur, w1, b1.reshape(1, _C))


def kernel(x, edge_index, w0, b0, conv_w, w1, b1):
    e = edge_index.reshape(2, _NW, _EPW)
    src = jnp.concatenate(
        [e[0], jnp.zeros((_NW, _PAD), jnp.int32)], axis=1).reshape(-1)
    dst = jnp.concatenate(
        [e[1], jnp.full((_NW, _PAD), _TRASH, jnp.int32)], axis=1).reshape(-1)
    h = _init_tc(x, w0, b0)
    x0 = h
    xcur = h
    for layer in range(_LAYERS):
        beta = math.log(1.0 / (layer + 1) + 1.0)
        parts = _segsum(xcur, src, dst)
        xcur = _layer_tc(parts, x0, xcur, conv_w[layer], beta)
    return _final_tc(xcur, w1, b1)
